# Initial kernel scaffold; baseline (speedup 1.0000x reference)
import jax
import jax.numpy as jnp
from jax.experimental import pallas as pl


def kernel(sessions, params):
    raise NotImplementedError("write your pallas kernel here")


# speedup vs baseline: 2.5617x; 2.5617x over previous
import functools

import jax
import jax.numpy as jnp
from jax import lax
from jax.experimental import pallas as pl
from jax.experimental.pallas import tpu as pltpu
from jax.experimental.pallas import tpu_sc as plsc

N_ITEMS = 100000
D = 128
L = 200
H = 4
DH = D // H
NB = 2
B = 1024

BT = 8
SC_CHUNK = 256


def _sc_gather(table, idx_flat):
    R = idx_flat.shape[0]
    info = plsc.get_sparse_core_info()
    nw = info.num_cores * info.num_subcores
    per_w = R // nw
    n_ch = per_w // SC_CHUNK
    mesh = plsc.VectorSubcoreMesh(core_axis_name="c", subcore_axis_name="s")

    @functools.partial(
        pl.kernel,
        mesh=mesh,
        out_type=jax.ShapeDtypeStruct((R, D), jnp.float32),
        scratch_types=[
            pltpu.VMEM((SC_CHUNK,), jnp.int32),
            pltpu.VMEM((SC_CHUNK, D), jnp.float32),
            pltpu.SemaphoreType.DMA,
        ],
    )
    def gather_kernel(table_hbm, idx_hbm, out_hbm, idx_v, rows_v, sem):
        wid = lax.axis_index("s") * info.num_cores + lax.axis_index("c")
        base = wid * per_w

        def body(i, carry):
            off = base + i * SC_CHUNK
            pltpu.sync_copy(idx_hbm.at[pl.ds(off, SC_CHUNK)], idx_v)
            pltpu.async_copy(table_hbm.at[idx_v], rows_v, sem).wait()
            pltpu.sync_copy(rows_v, out_hbm.at[pl.ds(off, SC_CHUNK)])
            return carry

        lax.fori_loop(0, n_ch, body, 0)

    return gather_kernel(table, idx_flat)


def _ln(x, g, b, eps=1e-5):
    m = jnp.mean(x, axis=-1, keepdims=True)
    c = x - m
    v = jnp.mean(c * c, axis=-1, keepdims=True)
    return c * jax.lax.rsqrt(v + eps) * g + b


def _tc_body(x_ref, pos_ref, wq_ref, wkv_ref, wo_ref, w1_ref, w2_ref,
             vec_ref, o_ref):
    x = x_ref[...].reshape(BT * L, D)
    pos = pos_ref[...]
    x = (x.reshape(BT, L, D) + pos[None]).reshape(BT * L, D)

    scale = 1.0 / (DH ** 0.5)
    row = lax.broadcasted_iota(jnp.int32, (L, L), 0)
    col = lax.broadcasted_iota(jnp.int32, (L, L), 1)
    allow = (col <= row)[None]

    for nb in range(NB):
        vbase = nb * 10
        q = _ln(x, vec_ref[vbase + 4], vec_ref[vbase + 5])
        Q = jnp.dot(q, wq_ref[nb], preferred_element_type=jnp.float32) \
            + vec_ref[vbase + 0]
        KV = jnp.dot(x, wkv_ref[nb], preferred_element_type=jnp.float32)
        K = KV[:, :D] + vec_ref[vbase + 1]
        V = KV[:, D:] + vec_ref[vbase + 2]

        outs = []
        for h in range(H):
            sl = slice(h * DH, (h + 1) * DH)
            Qh = Q[:, sl].reshape(BT, L, DH)
            Kh = K[:, sl].reshape(BT, L, DH)
            Vh = V[:, sl].reshape(BT, L, DH)
            s = lax.dot_general(
                Qh, Kh, (((2,), (2,)), ((0,), (0,))),
                preferred_element_type=jnp.float32) * scale
            s = jnp.where(allow, s, -1e30)
            s = s - jnp.max(s, axis=-1, keepdims=True)
            e = jnp.exp(s)
            p = e / jnp.sum(e, axis=-1, keepdims=True)
            oh = lax.dot_general(
                p, Vh, (((2,), (1,)), ((0,), (0,))),
                preferred_element_type=jnp.float32)
            outs.append(oh.reshape(BT * L, DH))
        ao = jnp.concatenate(outs, axis=-1)
        mha = jnp.dot(ao, wo_ref[nb], preferred_element_type=jnp.float32) \
            + vec_ref[vbase + 3]
        x = q + mha

        ff_in = _ln(x, vec_ref[vbase + 6], vec_ref[vbase + 7])
        h1 = jnp.maximum(
            jnp.dot(ff_in, w1_ref[nb], preferred_element_type=jnp.float32)
            + vec_ref[vbase + 8], 0.0)
        ff = jnp.dot(h1, w2_ref[nb], preferred_element_type=jnp.float32) \
            + vec_ref[vbase + 9]
        x = ff + ff_in

    out = _ln(x, vec_ref[NB * 10], vec_ref[NB * 10 + 1])
    o_ref[...] = out.reshape(BT, L, D)


def _tc_transformer(seqs, pos, wq, wkv, wo, w1, w2, vecs):
    grid = B // BT
    return pl.pallas_call(
        _tc_body,
        grid=(grid,),
        in_specs=[
            pl.BlockSpec((BT, L, D), lambda i: (i, 0, 0)),
            pl.BlockSpec((L, D), lambda i: (0, 0)),
            pl.BlockSpec((NB, D, D), lambda i: (0, 0, 0)),
            pl.BlockSpec((NB, D, 2 * D), lambda i: (0, 0, 0)),
            pl.BlockSpec((NB, D, D), lambda i: (0, 0, 0)),
            pl.BlockSpec((NB, D, D), lambda i: (0, 0, 0)),
            pl.BlockSpec((NB, D, D), lambda i: (0, 0, 0)),
            pl.BlockSpec((NB * 10 + 2, D), lambda i: (0, 0)),
        ],
        out_specs=pl.BlockSpec((BT, L, D), lambda i: (i, 0, 0)),
        out_shape=jax.ShapeDtypeStruct((B, L, D), jnp.float32),
    )(seqs, pos, wq, wkv, wo, w1, w2, vecs)


def _pack_params(params):
    wq_l, wkv_l, wo_l, w1_l, w2_l = [], [], [], [], []
    vec_rows = []
    for blk in params['blocks']:
        w = blk['in_proj_w']
        wq_l.append(w[:D].T)
        wkv_l.append(jnp.concatenate([w[D:2 * D].T, w[2 * D:].T], axis=1))
        wo_l.append(blk['out_proj_w'].T)
        w1_l.append(blk['ff_w1'].T)
        w2_l.append(blk['ff_w2'].T)
        b = blk['in_proj_b']
        vec_rows += [b[:D], b[D:2 * D], b[2 * D:], blk['out_proj_b'],
                     blk['q_ln_g'], blk['q_ln_b'], blk['ff_ln_g'],
                     blk['ff_ln_b'], blk['ff_b1'], blk['ff_b2']]
    vec_rows += [params['last_ln_g'], params['last_ln_b']]
    return (jnp.stack(wq_l), jnp.stack(wkv_l), jnp.stack(wo_l),
            jnp.stack(w1_l), jnp.stack(w2_l), jnp.stack(vec_rows))


def kernel(sessions, params):
    idx_flat = sessions.reshape(B * L).astype(jnp.int32)
    seqs = _sc_gather(params['item_emb'], idx_flat).reshape(B, L, D)
    pos = params['pos_emb'][::-1]
    wq, wkv, wo, w1, w2, vecs = _pack_params(params)
    return _tc_transformer(seqs, pos, wq, wkv, wo, w1, w2, vecs)
